# baseline (speedup 1.0000x reference)
import functools

import jax
import jax.numpy as jnp
from jax.experimental import pallas as pl
from jax.experimental.pallas import tpu as pltpu

N = 10000
E = 320000
EMB = 128
NG = 128
BLK = 1000
GRID = N // BLK
TBL = 256


def _embed_body(xc_ref, b_ref, tbl_ref, vnrow_ref, hl_ref, pooled_ref):
    i = pl.program_id(0)
    xc = xc_ref[0]
    cols = jax.lax.broadcasted_iota(jnp.int32, (BLK, TBL), 1)
    oh = ((cols == xc[:, 0:1]) | (cols == xc[:, 1:2]) | (cols == xc[:, 2:3]))
    oh = oh.astype(jnp.float32)
    hl = jnp.dot(oh, tbl_ref[...], preferred_element_type=jnp.float32, precision=jax.lax.Precision.HIGHEST)
    hl = hl + vnrow_ref[0][None, :]
    hl_ref[...] = hl
    seg = b_ref[0, 0]
    gcols = jax.lax.broadcasted_iota(jnp.int32, (BLK, NG), 1)
    ohb = (gcols == seg[:, None]).astype(jnp.float32)
    part = jax.lax.dot_general(ohb, hl, (((0,), (0,)), ((), ())),
                               preferred_element_type=jnp.float32, precision=jax.lax.Precision.HIGHEST)

    @pl.when(i == 0)
    def _():
        pooled_ref[...] = part

    @pl.when(i != 0)
    def _():
        pooled_ref[...] += part


def _embed_call(xc3, batch3, tbl, vnrow):
    return pl.pallas_call(
        _embed_body,
        grid=(GRID,),
        in_specs=[
            pl.BlockSpec((1, BLK, 3), lambda i: (i, 0, 0)),
            pl.BlockSpec((1, 1, BLK), lambda i: (i, 0, 0)),
            pl.BlockSpec((TBL, EMB), lambda i: (0, 0)),
            pl.BlockSpec((1, EMB), lambda i: (0, 0)),
        ],
        out_specs=[
            pl.BlockSpec((BLK, EMB), lambda i: (i, 0)),
            pl.BlockSpec((NG, EMB), lambda i: (0, 0)),
        ],
        out_shape=[
            jax.ShapeDtypeStruct((N, EMB), jnp.float32),
            jax.ShapeDtypeStruct((NG, EMB), jnp.float32),
        ],
    )(xc3, batch3, tbl, vnrow)


def _addvn_body(h_ref, b_ref, vn_ref, hl_ref, pooled_ref):
    i = pl.program_id(0)
    seg = b_ref[0, 0]
    gcols = jax.lax.broadcasted_iota(jnp.int32, (BLK, NG), 1)
    ohb = (gcols == seg[:, None]).astype(jnp.float32)
    hl = h_ref[...] + jnp.dot(ohb, vn_ref[...],
                              preferred_element_type=jnp.float32, precision=jax.lax.Precision.HIGHEST)
    hl_ref[...] = hl
    part = jax.lax.dot_general(ohb, hl, (((0,), (0,)), ((), ())),
                               preferred_element_type=jnp.float32, precision=jax.lax.Precision.HIGHEST)

    @pl.when(i == 0)
    def _():
        pooled_ref[...] = part

    @pl.when(i != 0)
    def _():
        pooled_ref[...] += part


def _addvn_call(h, batch3, vn):
    return pl.pallas_call(
        _addvn_body,
        grid=(GRID,),
        in_specs=[
            pl.BlockSpec((BLK, EMB), lambda i: (i, 0)),
            pl.BlockSpec((1, 1, BLK), lambda i: (i, 0, 0)),
            pl.BlockSpec((NG, EMB), lambda i: (0, 0)),
        ],
        out_specs=[
            pl.BlockSpec((BLK, EMB), lambda i: (i, 0)),
            pl.BlockSpec((NG, EMB), lambda i: (0, 0)),
        ],
        out_shape=[
            jax.ShapeDtypeStruct((N, EMB), jnp.float32),
            jax.ShapeDtypeStruct((NG, EMB), jnp.float32),
        ],
    )(h, batch3, vn)


def _mlp1_body(hl_ref, a0_ref, a1_ref, epsp1_ref, w1_ref, b1_ref,
               y1_ref, st_ref):
    i = pl.program_id(0)
    z = epsp1_ref[0, 0] * hl_ref[...] + a0_ref[...] + a1_ref[...]
    y1 = jnp.dot(z, w1_ref[...], preferred_element_type=jnp.float32, precision=jax.lax.Precision.DEFAULT)
    y1 = y1 + b1_ref[0][None, :]
    y1_ref[...] = y1
    s = jnp.sum(y1, axis=0)
    ss = jnp.sum(y1 * y1, axis=0)
    part = jnp.stack([s, ss])

    @pl.when(i == 0)
    def _():
        st_ref[...] = part

    @pl.when(i != 0)
    def _():
        st_ref[...] += part


def _mlp1_call(hl, a0, a1, epsp1, w1, b1):
    h2 = 2 * EMB
    return pl.pallas_call(
        _mlp1_body,
        grid=(GRID,),
        in_specs=[
            pl.BlockSpec((BLK, EMB), lambda i: (i, 0)),
            pl.BlockSpec((BLK, EMB), lambda i: (i, 0)),
            pl.BlockSpec((BLK, EMB), lambda i: (i, 0)),
            pl.BlockSpec((1, 1), lambda i: (0, 0)),
            pl.BlockSpec((EMB, h2), lambda i: (0, 0)),
            pl.BlockSpec((1, h2), lambda i: (0, 0)),
        ],
        out_specs=[
            pl.BlockSpec((BLK, h2), lambda i: (i, 0)),
            pl.BlockSpec((2, h2), lambda i: (0, 0)),
        ],
        out_shape=[
            jax.ShapeDtypeStruct((N, h2), jnp.float32),
            jax.ShapeDtypeStruct((2, h2), jnp.float32),
        ],
    )(hl, a0, a1, epsp1, w1, b1)


def _mlp2_body(y1_ref, st_ref, g1_ref, bb1_ref, w2_ref, b2_ref,
               y2_ref, st2_ref):
    i = pl.program_id(0)
    s = st_ref[0]
    ss = st_ref[1]
    m = s / N
    v = ss / N - m * m
    inv = 1.0 / jnp.sqrt(v + 1e-5)
    t = g1_ref[0][None, :] * (y1_ref[...] - m[None, :]) * inv[None, :] \
        + bb1_ref[0][None, :]
    t = jax.nn.relu(t)
    y2 = jnp.dot(t, w2_ref[...], preferred_element_type=jnp.float32, precision=jax.lax.Precision.DEFAULT)
    y2 = y2 + b2_ref[0][None, :]
    y2_ref[...] = y2
    part = jnp.stack([jnp.sum(y2, axis=0), jnp.sum(y2 * y2, axis=0)])

    @pl.when(i == 0)
    def _():
        st2_ref[...] = part

    @pl.when(i != 0)
    def _():
        st2_ref[...] += part


def _mlp2_call(y1, st, g1, bb1, w2, b2):
    h2 = 2 * EMB
    return pl.pallas_call(
        _mlp2_body,
        grid=(GRID,),
        in_specs=[
            pl.BlockSpec((BLK, h2), lambda i: (i, 0)),
            pl.BlockSpec((2, h2), lambda i: (0, 0)),
            pl.BlockSpec((1, h2), lambda i: (0, 0)),
            pl.BlockSpec((1, h2), lambda i: (0, 0)),
            pl.BlockSpec((h2, EMB), lambda i: (0, 0)),
            pl.BlockSpec((1, EMB), lambda i: (0, 0)),
        ],
        out_specs=[
            pl.BlockSpec((BLK, EMB), lambda i: (i, 0)),
            pl.BlockSpec((2, EMB), lambda i: (0, 0)),
        ],
        out_shape=[
            jax.ShapeDtypeStruct((N, EMB), jnp.float32),
            jax.ShapeDtypeStruct((2, EMB), jnp.float32),
        ],
    )(y1, st, g1, bb1, w2, b2)


def _bnout_body(relu, y2_ref, st_ref, g_ref, b_ref, h_ref):
    s = st_ref[0]
    ss = st_ref[1]
    m = s / N
    v = ss / N - m * m
    inv = 1.0 / jnp.sqrt(v + 1e-5)
    h = g_ref[0][None, :] * (y2_ref[...] - m[None, :]) * inv[None, :] \
        + b_ref[0][None, :]
    if relu:
        h = jax.nn.relu(h)
    h_ref[...] = h


def _bnout_call(y2, st, g, b, relu):
    return pl.pallas_call(
        functools.partial(_bnout_body, relu),
        grid=(GRID,),
        in_specs=[
            pl.BlockSpec((BLK, EMB), lambda i: (i, 0)),
            pl.BlockSpec((2, EMB), lambda i: (0, 0)),
            pl.BlockSpec((1, EMB), lambda i: (0, 0)),
            pl.BlockSpec((1, EMB), lambda i: (0, 0)),
        ],
        out_specs=pl.BlockSpec((BLK, EMB), lambda i: (i, 0)),
        out_shape=jax.ShapeDtypeStruct((N, EMB), jnp.float32),
    )(y2, st, g, b)


def _vnmlp_body(pooled_ref, vn_ref, w1_ref, b1_ref, g1_ref, bb1_ref,
                w2_ref, b2_ref, g2_ref, bb2_ref, out_ref):
    p = pooled_ref[...] + vn_ref[...]
    t = jnp.dot(p, w1_ref[...], preferred_element_type=jnp.float32, precision=jax.lax.Precision.DEFAULT)
    t = t + b1_ref[0][None, :]
    m = jnp.mean(t, axis=0)
    v = jnp.mean(t * t, axis=0) - m * m
    t = g1_ref[0] * (t - m[None, :]) / jnp.sqrt(v + 1e-5)[None, :] \
        + bb1_ref[0][None, :]
    t = jax.nn.relu(t)
    u = jnp.dot(t, w2_ref[...], preferred_element_type=jnp.float32, precision=jax.lax.Precision.DEFAULT)
    u = u + b2_ref[0][None, :]
    m2 = jnp.mean(u, axis=0)
    v2 = jnp.mean(u * u, axis=0) - m2 * m2
    u = g2_ref[0] * (u - m2[None, :]) / jnp.sqrt(v2 + 1e-5)[None, :] \
        + bb2_ref[0][None, :]
    out_ref[...] = jax.nn.relu(u)


def _vnmlp_call(pooled, vn, mp):
    h2 = 2 * EMB
    args = (pooled, vn, mp['W1'], mp['b1'].reshape(1, h2),
            mp['g1'].reshape(1, h2), mp['bb1'].reshape(1, h2),
            mp['W2'], mp['b2'].reshape(1, EMB),
            mp['g2'].reshape(1, EMB), mp['bb2'].reshape(1, EMB))
    return pl.pallas_call(
        _vnmlp_body,
        out_shape=jax.ShapeDtypeStruct((NG, EMB), jnp.float32),
    )(*args)


def _edge_stage(hl, src, dst, edge_attr, cp):
    ee = edge_attr @ cp['We'] + cp['be']
    msg = jax.nn.relu(hl[src] + ee)
    agg = jax.ops.segment_sum(msg, dst, num_segments=N)
    zero = jnp.zeros((N, EMB), jnp.float32)
    return agg, zero


def kernel(x, edge_index, edge_attr, node_depth, batch, params):
    p = params
    nd = jnp.clip(node_depth[:, 0], 0, 20).astype(jnp.int32)
    xc = jnp.stack([x[:, 0].astype(jnp.int32),
                    100 + x[:, 1].astype(jnp.int32),
                    200 + nd], axis=1)
    xc3 = xc.reshape(GRID, BLK, 3)
    batch3 = batch.astype(jnp.int32).reshape(GRID, 1, BLK)
    tbl = jnp.zeros((TBL, EMB), jnp.float32)
    tbl = tbl.at[0:100].set(p['type_emb'])
    tbl = tbl.at[100:200].set(p['attr_emb'])
    tbl = tbl.at[200:221].set(p['depth_emb'])
    vnrow = p['vn_emb'][0:1]
    src = edge_index[0].astype(jnp.int32)
    dst = edge_index[1].astype(jnp.int32)

    vn = jnp.broadcast_to(vnrow, (NG, EMB))
    hl, pooled = _embed_call(xc3, batch3, tbl, vnrow)

    out = None
    for layer in range(3):
        cp = p['convs'][layer]
        a0, a1 = _edge_stage(hl, src, dst, edge_attr, cp)
        epsp1 = (1.0 + cp['eps']).reshape(1, 1)
        y1, st1 = _mlp1_call(hl, a0, a1, epsp1, cp['W1'],
                             cp['b1'].reshape(1, 2 * EMB))
        y2, st2 = _mlp2_call(y1, st1, cp['g1'].reshape(1, 2 * EMB),
                             cp['bb1'].reshape(1, 2 * EMB), cp['W2'],
                             cp['b2'].reshape(1, EMB))
        g = p['bn_g'][layer].reshape(1, EMB)
        b = p['bn_b'][layer].reshape(1, EMB)
        if layer < 2:
            h = _bnout_call(y2, st2, g, b, relu=True)
            vn = _vnmlp_call(pooled, vn, p['vn_mlps'][layer])
            hl, pooled = _addvn_call(h, batch3, vn)
        else:
            out = _bnout_call(y2, st2, g, b, relu=False)
    return out


# speedup vs baseline: 2.4352x; 2.4352x over previous
import functools

import jax
import jax.numpy as jnp
from jax import lax
from jax.experimental import pallas as pl
from jax.experimental.pallas import tpu as pltpu
from jax.experimental.pallas import tpu_sc as plsc

N = 10000
E = 320000
EMB = 128
NG = 128
BLK = 1000
GRID = N // BLK
TBL = 256


def _embed_body(xc_ref, b_ref, tbl_ref, vnrow_ref, hl_ref, pooled_ref):
    i = pl.program_id(0)
    xc = xc_ref[0]
    cols = jax.lax.broadcasted_iota(jnp.int32, (BLK, TBL), 1)
    oh = ((cols == xc[:, 0:1]) | (cols == xc[:, 1:2]) | (cols == xc[:, 2:3]))
    oh = oh.astype(jnp.float32)
    hl = jnp.dot(oh, tbl_ref[...], preferred_element_type=jnp.float32, precision=jax.lax.Precision.HIGHEST)
    hl = hl + vnrow_ref[0][None, :]
    hl_ref[...] = hl
    seg = b_ref[0, 0]
    gcols = jax.lax.broadcasted_iota(jnp.int32, (BLK, NG), 1)
    ohb = (gcols == seg[:, None]).astype(jnp.float32)
    part = jax.lax.dot_general(ohb, hl, (((0,), (0,)), ((), ())),
                               preferred_element_type=jnp.float32, precision=jax.lax.Precision.HIGHEST)

    @pl.when(i == 0)
    def _():
        pooled_ref[...] = part

    @pl.when(i != 0)
    def _():
        pooled_ref[...] += part


def _embed_call(xc3, batch3, tbl, vnrow):
    return pl.pallas_call(
        _embed_body,
        grid=(GRID,),
        in_specs=[
            pl.BlockSpec((1, BLK, 3), lambda i: (i, 0, 0)),
            pl.BlockSpec((1, 1, BLK), lambda i: (i, 0, 0)),
            pl.BlockSpec((TBL, EMB), lambda i: (0, 0)),
            pl.BlockSpec((1, EMB), lambda i: (0, 0)),
        ],
        out_specs=[
            pl.BlockSpec((BLK, EMB), lambda i: (i, 0)),
            pl.BlockSpec((NG, EMB), lambda i: (0, 0)),
        ],
        out_shape=[
            jax.ShapeDtypeStruct((N, EMB), jnp.float32),
            jax.ShapeDtypeStruct((NG, EMB), jnp.float32),
        ],
    )(xc3, batch3, tbl, vnrow)


def _addvn_body(h_ref, b_ref, vn_ref, hl_ref, pooled_ref):
    i = pl.program_id(0)
    seg = b_ref[0, 0]
    gcols = jax.lax.broadcasted_iota(jnp.int32, (BLK, NG), 1)
    ohb = (gcols == seg[:, None]).astype(jnp.float32)
    hl = h_ref[...] + jnp.dot(ohb, vn_ref[...],
                              preferred_element_type=jnp.float32, precision=jax.lax.Precision.HIGHEST)
    hl_ref[...] = hl
    part = jax.lax.dot_general(ohb, hl, (((0,), (0,)), ((), ())),
                               preferred_element_type=jnp.float32, precision=jax.lax.Precision.HIGHEST)

    @pl.when(i == 0)
    def _():
        pooled_ref[...] = part

    @pl.when(i != 0)
    def _():
        pooled_ref[...] += part


def _addvn_call(h, batch3, vn):
    return pl.pallas_call(
        _addvn_body,
        grid=(GRID,),
        in_specs=[
            pl.BlockSpec((BLK, EMB), lambda i: (i, 0)),
            pl.BlockSpec((1, 1, BLK), lambda i: (i, 0, 0)),
            pl.BlockSpec((NG, EMB), lambda i: (0, 0)),
        ],
        out_specs=[
            pl.BlockSpec((BLK, EMB), lambda i: (i, 0)),
            pl.BlockSpec((NG, EMB), lambda i: (0, 0)),
        ],
        out_shape=[
            jax.ShapeDtypeStruct((N, EMB), jnp.float32),
            jax.ShapeDtypeStruct((NG, EMB), jnp.float32),
        ],
    )(h, batch3, vn)


def _mlp1_body(hl_ref, a0_ref, a1_ref, epsp1_ref, w1_ref, b1_ref,
               y1_ref, st_ref):
    i = pl.program_id(0)
    z = epsp1_ref[0, 0] * hl_ref[...] + a0_ref[0] + a1_ref[0]
    y1 = jnp.dot(z, w1_ref[...], preferred_element_type=jnp.float32, precision=jax.lax.Precision.DEFAULT)
    y1 = y1 + b1_ref[0][None, :]
    y1_ref[...] = y1
    s = jnp.sum(y1, axis=0)
    ss = jnp.sum(y1 * y1, axis=0)
    part = jnp.stack([s, ss])

    @pl.when(i == 0)
    def _():
        st_ref[...] = part

    @pl.when(i != 0)
    def _():
        st_ref[...] += part


def _mlp1_call(hl, agg2, epsp1, w1, b1):
    h2 = 2 * EMB
    return pl.pallas_call(
        _mlp1_body,
        grid=(GRID,),
        in_specs=[
            pl.BlockSpec((BLK, EMB), lambda i: (i, 0)),
            pl.BlockSpec((1, BLK, EMB), lambda i: (0, i, 0)),
            pl.BlockSpec((1, BLK, EMB), lambda i: (1, i, 0)),
            pl.BlockSpec((1, 1), lambda i: (0, 0)),
            pl.BlockSpec((EMB, h2), lambda i: (0, 0)),
            pl.BlockSpec((1, h2), lambda i: (0, 0)),
        ],
        out_specs=[
            pl.BlockSpec((BLK, h2), lambda i: (i, 0)),
            pl.BlockSpec((2, h2), lambda i: (0, 0)),
        ],
        out_shape=[
            jax.ShapeDtypeStruct((N, h2), jnp.float32),
            jax.ShapeDtypeStruct((2, h2), jnp.float32),
        ],
    )(hl, agg2, agg2, epsp1, w1, b1)


def _mlp2_body(y1_ref, st_ref, g1_ref, bb1_ref, w2_ref, b2_ref,
               y2_ref, st2_ref):
    i = pl.program_id(0)
    s = st_ref[0]
    ss = st_ref[1]
    m = s / N
    v = ss / N - m * m
    inv = 1.0 / jnp.sqrt(v + 1e-5)
    t = g1_ref[0][None, :] * (y1_ref[...] - m[None, :]) * inv[None, :] \
        + bb1_ref[0][None, :]
    t = jax.nn.relu(t)
    y2 = jnp.dot(t, w2_ref[...], preferred_element_type=jnp.float32, precision=jax.lax.Precision.DEFAULT)
    y2 = y2 + b2_ref[0][None, :]
    y2_ref[...] = y2
    part = jnp.stack([jnp.sum(y2, axis=0), jnp.sum(y2 * y2, axis=0)])

    @pl.when(i == 0)
    def _():
        st2_ref[...] = part

    @pl.when(i != 0)
    def _():
        st2_ref[...] += part


def _mlp2_call(y1, st, g1, bb1, w2, b2):
    h2 = 2 * EMB
    return pl.pallas_call(
        _mlp2_body,
        grid=(GRID,),
        in_specs=[
            pl.BlockSpec((BLK, h2), lambda i: (i, 0)),
            pl.BlockSpec((2, h2), lambda i: (0, 0)),
            pl.BlockSpec((1, h2), lambda i: (0, 0)),
            pl.BlockSpec((1, h2), lambda i: (0, 0)),
            pl.BlockSpec((h2, EMB), lambda i: (0, 0)),
            pl.BlockSpec((1, EMB), lambda i: (0, 0)),
        ],
        out_specs=[
            pl.BlockSpec((BLK, EMB), lambda i: (i, 0)),
            pl.BlockSpec((2, EMB), lambda i: (0, 0)),
        ],
        out_shape=[
            jax.ShapeDtypeStruct((N, EMB), jnp.float32),
            jax.ShapeDtypeStruct((2, EMB), jnp.float32),
        ],
    )(y1, st, g1, bb1, w2, b2)


def _bnout_body(relu, y2_ref, st_ref, g_ref, b_ref, h_ref):
    s = st_ref[0]
    ss = st_ref[1]
    m = s / N
    v = ss / N - m * m
    inv = 1.0 / jnp.sqrt(v + 1e-5)
    h = g_ref[0][None, :] * (y2_ref[...] - m[None, :]) * inv[None, :] \
        + b_ref[0][None, :]
    if relu:
        h = jax.nn.relu(h)
    h_ref[...] = h


def _bnout_call(y2, st, g, b, relu):
    return pl.pallas_call(
        functools.partial(_bnout_body, relu),
        grid=(GRID,),
        in_specs=[
            pl.BlockSpec((BLK, EMB), lambda i: (i, 0)),
            pl.BlockSpec((2, EMB), lambda i: (0, 0)),
            pl.BlockSpec((1, EMB), lambda i: (0, 0)),
            pl.BlockSpec((1, EMB), lambda i: (0, 0)),
        ],
        out_specs=pl.BlockSpec((BLK, EMB), lambda i: (i, 0)),
        out_shape=jax.ShapeDtypeStruct((N, EMB), jnp.float32),
    )(y2, st, g, b)


def _vnmlp_body(pooled_ref, vn_ref, w1_ref, b1_ref, g1_ref, bb1_ref,
                w2_ref, b2_ref, g2_ref, bb2_ref, out_ref):
    p = pooled_ref[...] + vn_ref[...]
    t = jnp.dot(p, w1_ref[...], preferred_element_type=jnp.float32, precision=jax.lax.Precision.DEFAULT)
    t = t + b1_ref[0][None, :]
    m = jnp.mean(t, axis=0)
    v = jnp.mean(t * t, axis=0) - m * m
    t = g1_ref[0] * (t - m[None, :]) / jnp.sqrt(v + 1e-5)[None, :] \
        + bb1_ref[0][None, :]
    t = jax.nn.relu(t)
    u = jnp.dot(t, w2_ref[...], preferred_element_type=jnp.float32, precision=jax.lax.Precision.DEFAULT)
    u = u + b2_ref[0][None, :]
    m2 = jnp.mean(u, axis=0)
    v2 = jnp.mean(u * u, axis=0) - m2 * m2
    u = g2_ref[0] * (u - m2[None, :]) / jnp.sqrt(v2 + 1e-5)[None, :] \
        + bb2_ref[0][None, :]
    out_ref[...] = jax.nn.relu(u)


def _vnmlp_call(pooled, vn, mp):
    h2 = 2 * EMB
    args = (pooled, vn, mp['W1'], mp['b1'].reshape(1, h2),
            mp['g1'].reshape(1, h2), mp['bb1'].reshape(1, h2),
            mp['W2'], mp['b2'].reshape(1, EMB),
            mp['g2'].reshape(1, EMB), mp['bb2'].reshape(1, EMB))
    return pl.pallas_call(
        _vnmlp_body,
        out_shape=jax.ShapeDtypeStruct((NG, EMB), jnp.float32),
    )(*args)


def _bf16r(v):
    u = lax.bitcast_convert_type(v, jnp.int32)
    r = (u + jnp.int32(0x7FFF) + ((u >> 16) & 1)) & jnp.int32(-65536)
    return lax.bitcast_convert_type(r, jnp.float32)


NW = 32
EPW = E // NW
CH = 80
NCH = EPW // CH
NP = 10240
ZR = NP // 16


def _edge_body(hl, src3, dst3, a3, b3, wep, zrows, out,
               src_v, dst_v, a_v, b_v, rows_v, wep_v, agg_sh):
    c = lax.axis_index("c")
    s = lax.axis_index("s")
    wid = c * 16 + s
    pltpu.sync_copy(wep, wep_v)
    for j in range(16):
        wep_v[pl.ds(16 * j, 16)] = _bf16r(wep_v[pl.ds(16 * j, 16)])
    pltpu.sync_copy(zrows.at[pl.ds(0, ZR)], agg_sh.at[pl.ds(s * ZR, ZR)])
    plsc.subcore_barrier()

    def chunk(k, carry):
        pltpu.sync_copy(src3.at[wid, k], src_v)
        pltpu.sync_copy(dst3.at[wid, k], dst_v)
        pltpu.sync_copy(a3.at[wid, k], a_v)
        pltpu.sync_copy(b3.at[wid, k], b_v)
        pltpu.sync_copy(hl.at[src_v], rows_v)

        def group(g, carry2):
            av = _bf16r(a_v[pl.ds(g * 16, 16)])
            bv = _bf16r(b_v[pl.ds(g * 16, 16)])
            dn = lax.GatherDimensionNumbers(
                offset_dims=(), collapsed_slice_dims=(0,),
                start_index_map=(0,))

            def _bcast(vec, l):
                idx = jnp.full((16, 1), l, jnp.int32)
                return lax.gather(
                    vec, idx, dn, slice_sizes=(1,),
                    mode=lax.GatherScatterMode.PROMISE_IN_BOUNDS)

            abc = [_bcast(av, l) for l in range(16)]
            bbc = [_bcast(bv, l) for l in range(16)]
            for j in range(8):
                w0 = wep_v[pl.ds(16 * j, 16)]
                w1 = wep_v[pl.ds(128 + 16 * j, 16)]
                bj = wep_v[pl.ds(256 + 16 * j, 16)]
                for l in range(16):
                    e = g * 16 + l
                    r = rows_v[e, pl.ds(16 * j, 16)]
                    rows_v[e, pl.ds(16 * j, 16)] = jnp.maximum(
                        r + (abc[l] * w0 + bbc[l] * w1 + bj), 0.0)
            return carry2

        lax.fori_loop(0, CH // 16, group, 0)
        pltpu.sync_copy(rows_v, agg_sh.at[dst_v], add=True)
        return carry

    lax.fori_loop(0, NCH, chunk, 0)
    plsc.subcore_barrier()
    pltpu.sync_copy(agg_sh.at[pl.ds(s * ZR, ZR)],
                    out.at[c, pl.ds(s * ZR, ZR)])


def _edge_call(hl, src3, dst3, a3, b3, wep, zrows):
    mesh = plsc.VectorSubcoreMesh(core_axis_name="c", subcore_axis_name="s")
    f = pl.kernel(
        _edge_body,
        mesh=mesh,
        out_type=jax.ShapeDtypeStruct((2, NP, EMB), jnp.float32),
        scratch_types=[
            pltpu.VMEM((CH,), jnp.int32),
            pltpu.VMEM((CH,), jnp.int32),
            pltpu.VMEM((CH,), jnp.float32),
            pltpu.VMEM((CH,), jnp.float32),
            pltpu.VMEM((CH, EMB), jnp.float32),
            pltpu.VMEM((3 * EMB,), jnp.float32),
            pltpu.VMEM_SHARED((NP, EMB), jnp.float32),
        ],
    )
    return f(hl, src3, dst3, a3, b3, wep, zrows)


def kernel(x, edge_index, edge_attr, node_depth, batch, params):
    p = params
    nd = jnp.clip(node_depth[:, 0], 0, 20).astype(jnp.int32)
    xc = jnp.stack([x[:, 0].astype(jnp.int32),
                    100 + x[:, 1].astype(jnp.int32),
                    200 + nd], axis=1)
    xc3 = xc.reshape(GRID, BLK, 3)
    batch3 = batch.astype(jnp.int32).reshape(GRID, 1, BLK)
    tbl = jnp.zeros((TBL, EMB), jnp.float32)
    tbl = tbl.at[0:100].set(p['type_emb'])
    tbl = tbl.at[100:200].set(p['attr_emb'])
    tbl = tbl.at[200:221].set(p['depth_emb'])
    vnrow = p['vn_emb'][0:1]
    src3 = edge_index[0].astype(jnp.int32).reshape(NW, NCH, CH)
    dst3 = edge_index[1].astype(jnp.int32).reshape(NW, NCH, CH)
    eab = edge_attr
    a3 = eab[:, 0].reshape(NW, NCH, CH)
    b3 = eab[:, 1].reshape(NW, NCH, CH)
    zrows = jnp.zeros((ZR, EMB), jnp.float32)

    vn = jnp.broadcast_to(vnrow, (NG, EMB))
    hl, pooled = _embed_call(xc3, batch3, tbl, vnrow)

    out = None
    for layer in range(3):
        cp = p['convs'][layer]
        web = cp['We']
        wep = jnp.concatenate([web[0], web[1], cp['be']])
        agg2 = _edge_call(hl, src3, dst3, a3, b3, wep, zrows)
        epsp1 = (1.0 + cp['eps']).reshape(1, 1)
        y1, st1 = _mlp1_call(hl, agg2, epsp1, cp['W1'],
                             cp['b1'].reshape(1, 2 * EMB))
        y2, st2 = _mlp2_call(y1, st1, cp['g1'].reshape(1, 2 * EMB),
                             cp['bb1'].reshape(1, 2 * EMB), cp['W2'],
                             cp['b2'].reshape(1, EMB))
        g = p['bn_g'][layer].reshape(1, EMB)
        b = p['bn_b'][layer].reshape(1, EMB)
        if layer < 2:
            h = _bnout_call(y2, st2, g, b, relu=True)
            vn = _vnmlp_call(pooled, vn, p['vn_mlps'][layer])
            hl, pooled = _addvn_call(h, batch3, vn)
        else:
            out = _bnout_call(y2, st2, g, b, relu=False)
    return out


# speedup vs baseline: 4.3335x; 1.7795x over previous
import functools

import jax
import jax.numpy as jnp
from jax import lax
from jax.experimental import pallas as pl
from jax.experimental.pallas import tpu as pltpu
from jax.experimental.pallas import tpu_sc as plsc

N = 10000
E = 320000
EMB = 128
NG = 128
BLK = 1000
GRID = N // BLK
TBL = 256


def _embed_body(xc_ref, b_ref, tbl_ref, vnrow_ref, hl_ref, pooled_ref):
    i = pl.program_id(0)
    xc = xc_ref[0]
    cols = jax.lax.broadcasted_iota(jnp.int32, (BLK, TBL), 1)
    oh = ((cols == xc[:, 0:1]) | (cols == xc[:, 1:2]) | (cols == xc[:, 2:3]))
    oh = oh.astype(jnp.float32)
    hl = jnp.dot(oh, tbl_ref[...], preferred_element_type=jnp.float32, precision=jax.lax.Precision.HIGHEST)
    hl = hl + vnrow_ref[0][None, :]
    hl_ref[...] = hl
    seg = b_ref[0, 0]
    gcols = jax.lax.broadcasted_iota(jnp.int32, (BLK, NG), 1)
    ohb = (gcols == seg[:, None]).astype(jnp.float32)
    part = jax.lax.dot_general(ohb, hl, (((0,), (0,)), ((), ())),
                               preferred_element_type=jnp.float32, precision=jax.lax.Precision.HIGHEST)

    @pl.when(i == 0)
    def _():
        pooled_ref[...] = part

    @pl.when(i != 0)
    def _():
        pooled_ref[...] += part


def _embed_call(xc3, batch3, tbl, vnrow):
    return pl.pallas_call(
        _embed_body,
        grid=(GRID,),
        in_specs=[
            pl.BlockSpec((1, BLK, 3), lambda i: (i, 0, 0)),
            pl.BlockSpec((1, 1, BLK), lambda i: (i, 0, 0)),
            pl.BlockSpec((TBL, EMB), lambda i: (0, 0)),
            pl.BlockSpec((1, EMB), lambda i: (0, 0)),
        ],
        out_specs=[
            pl.BlockSpec((BLK, EMB), lambda i: (i, 0)),
            pl.BlockSpec((NG, EMB), lambda i: (0, 0)),
        ],
        out_shape=[
            jax.ShapeDtypeStruct((N, EMB), jnp.float32),
            jax.ShapeDtypeStruct((NG, EMB), jnp.float32),
        ],
    )(xc3, batch3, tbl, vnrow)


def _addvn_body(h_ref, b_ref, vn_ref, hl_ref, pooled_ref):
    i = pl.program_id(0)
    seg = b_ref[0, 0]
    gcols = jax.lax.broadcasted_iota(jnp.int32, (BLK, NG), 1)
    ohb = (gcols == seg[:, None]).astype(jnp.float32)
    hl = h_ref[...] + jnp.dot(ohb, vn_ref[...],
                              preferred_element_type=jnp.float32, precision=jax.lax.Precision.HIGHEST)
    hl_ref[...] = hl
    part = jax.lax.dot_general(ohb, hl, (((0,), (0,)), ((), ())),
                               preferred_element_type=jnp.float32, precision=jax.lax.Precision.HIGHEST)

    @pl.when(i == 0)
    def _():
        pooled_ref[...] = part

    @pl.when(i != 0)
    def _():
        pooled_ref[...] += part


def _addvn_call(h, batch3, vn):
    return pl.pallas_call(
        _addvn_body,
        grid=(GRID,),
        in_specs=[
            pl.BlockSpec((BLK, EMB), lambda i: (i, 0)),
            pl.BlockSpec((1, 1, BLK), lambda i: (i, 0, 0)),
            pl.BlockSpec((NG, EMB), lambda i: (0, 0)),
        ],
        out_specs=[
            pl.BlockSpec((BLK, EMB), lambda i: (i, 0)),
            pl.BlockSpec((NG, EMB), lambda i: (0, 0)),
        ],
        out_shape=[
            jax.ShapeDtypeStruct((N, EMB), jnp.float32),
            jax.ShapeDtypeStruct((NG, EMB), jnp.float32),
        ],
    )(h, batch3, vn)


def _mlp1_body(hl_ref, a0_ref, a1_ref, epsp1_ref, w1_ref, b1_ref,
               y1_ref, st_ref):
    i = pl.program_id(0)
    z = epsp1_ref[0, 0] * hl_ref[...] + a0_ref[0] + a1_ref[0]
    y1 = jnp.dot(z, w1_ref[...], preferred_element_type=jnp.float32, precision=jax.lax.Precision.DEFAULT)
    y1 = y1 + b1_ref[0][None, :]
    y1_ref[...] = y1
    s = jnp.sum(y1, axis=0)
    ss = jnp.sum(y1 * y1, axis=0)
    part = jnp.stack([s, ss])

    @pl.when(i == 0)
    def _():
        st_ref[...] = part

    @pl.when(i != 0)
    def _():
        st_ref[...] += part


def _mlp1_call(hl, agg2, epsp1, w1, b1):
    h2 = 2 * EMB
    return pl.pallas_call(
        _mlp1_body,
        grid=(GRID,),
        in_specs=[
            pl.BlockSpec((BLK, EMB), lambda i: (i, 0)),
            pl.BlockSpec((1, BLK, EMB), lambda i: (0, i, 0)),
            pl.BlockSpec((1, BLK, EMB), lambda i: (1, i, 0)),
            pl.BlockSpec((1, 1), lambda i: (0, 0)),
            pl.BlockSpec((EMB, h2), lambda i: (0, 0)),
            pl.BlockSpec((1, h2), lambda i: (0, 0)),
        ],
        out_specs=[
            pl.BlockSpec((BLK, h2), lambda i: (i, 0)),
            pl.BlockSpec((2, h2), lambda i: (0, 0)),
        ],
        out_shape=[
            jax.ShapeDtypeStruct((N, h2), jnp.float32),
            jax.ShapeDtypeStruct((2, h2), jnp.float32),
        ],
    )(hl, agg2, agg2, epsp1, w1, b1)


def _mlp2_body(y1_ref, st_ref, g1_ref, bb1_ref, w2_ref, b2_ref,
               y2_ref, st2_ref):
    i = pl.program_id(0)
    s = st_ref[0]
    ss = st_ref[1]
    m = s / N
    v = ss / N - m * m
    inv = 1.0 / jnp.sqrt(v + 1e-5)
    t = g1_ref[0][None, :] * (y1_ref[...] - m[None, :]) * inv[None, :] \
        + bb1_ref[0][None, :]
    t = jax.nn.relu(t)
    y2 = jnp.dot(t, w2_ref[...], preferred_element_type=jnp.float32, precision=jax.lax.Precision.DEFAULT)
    y2 = y2 + b2_ref[0][None, :]
    y2_ref[...] = y2
    part = jnp.stack([jnp.sum(y2, axis=0), jnp.sum(y2 * y2, axis=0)])

    @pl.when(i == 0)
    def _():
        st2_ref[...] = part

    @pl.when(i != 0)
    def _():
        st2_ref[...] += part


def _mlp2_call(y1, st, g1, bb1, w2, b2):
    h2 = 2 * EMB
    return pl.pallas_call(
        _mlp2_body,
        grid=(GRID,),
        in_specs=[
            pl.BlockSpec((BLK, h2), lambda i: (i, 0)),
            pl.BlockSpec((2, h2), lambda i: (0, 0)),
            pl.BlockSpec((1, h2), lambda i: (0, 0)),
            pl.BlockSpec((1, h2), lambda i: (0, 0)),
            pl.BlockSpec((h2, EMB), lambda i: (0, 0)),
            pl.BlockSpec((1, EMB), lambda i: (0, 0)),
        ],
        out_specs=[
            pl.BlockSpec((BLK, EMB), lambda i: (i, 0)),
            pl.BlockSpec((2, EMB), lambda i: (0, 0)),
        ],
        out_shape=[
            jax.ShapeDtypeStruct((N, EMB), jnp.float32),
            jax.ShapeDtypeStruct((2, EMB), jnp.float32),
        ],
    )(y1, st, g1, bb1, w2, b2)


def _bnout_body(relu, y2_ref, st_ref, g_ref, b_ref, h_ref):
    s = st_ref[0]
    ss = st_ref[1]
    m = s / N
    v = ss / N - m * m
    inv = 1.0 / jnp.sqrt(v + 1e-5)
    h = g_ref[0][None, :] * (y2_ref[...] - m[None, :]) * inv[None, :] \
        + b_ref[0][None, :]
    if relu:
        h = jax.nn.relu(h)
    h_ref[...] = h


def _bnout_call(y2, st, g, b, relu):
    return pl.pallas_call(
        functools.partial(_bnout_body, relu),
        grid=(GRID,),
        in_specs=[
            pl.BlockSpec((BLK, EMB), lambda i: (i, 0)),
            pl.BlockSpec((2, EMB), lambda i: (0, 0)),
            pl.BlockSpec((1, EMB), lambda i: (0, 0)),
            pl.BlockSpec((1, EMB), lambda i: (0, 0)),
        ],
        out_specs=pl.BlockSpec((BLK, EMB), lambda i: (i, 0)),
        out_shape=jax.ShapeDtypeStruct((N, EMB), jnp.float32),
    )(y2, st, g, b)


def _vnmlp_body(pooled_ref, vn_ref, w1_ref, b1_ref, g1_ref, bb1_ref,
                w2_ref, b2_ref, g2_ref, bb2_ref, out_ref):
    p = pooled_ref[...] + vn_ref[...]
    t = jnp.dot(p, w1_ref[...], preferred_element_type=jnp.float32, precision=jax.lax.Precision.DEFAULT)
    t = t + b1_ref[0][None, :]
    m = jnp.mean(t, axis=0)
    v = jnp.mean(t * t, axis=0) - m * m
    t = g1_ref[0] * (t - m[None, :]) / jnp.sqrt(v + 1e-5)[None, :] \
        + bb1_ref[0][None, :]
    t = jax.nn.relu(t)
    u = jnp.dot(t, w2_ref[...], preferred_element_type=jnp.float32, precision=jax.lax.Precision.DEFAULT)
    u = u + b2_ref[0][None, :]
    m2 = jnp.mean(u, axis=0)
    v2 = jnp.mean(u * u, axis=0) - m2 * m2
    u = g2_ref[0] * (u - m2[None, :]) / jnp.sqrt(v2 + 1e-5)[None, :] \
        + bb2_ref[0][None, :]
    out_ref[...] = jax.nn.relu(u)


def _vnmlp_call(pooled, vn, mp):
    h2 = 2 * EMB
    args = (pooled, vn, mp['W1'], mp['b1'].reshape(1, h2),
            mp['g1'].reshape(1, h2), mp['bb1'].reshape(1, h2),
            mp['W2'], mp['b2'].reshape(1, EMB),
            mp['g2'].reshape(1, EMB), mp['bb2'].reshape(1, EMB))
    return pl.pallas_call(
        _vnmlp_body,
        out_shape=jax.ShapeDtypeStruct((NG, EMB), jnp.float32),
    )(*args)


def _bf16r(v):
    u = lax.bitcast_convert_type(v, jnp.int32)
    r = (u + jnp.int32(0x7FFF) + ((u >> 16) & 1)) & jnp.int32(-65536)
    return lax.bitcast_convert_type(r, jnp.float32)


NW = 32
EPW = E // NW
CH = 80
NCH = EPW // CH
NP = 10240
ZR = NP // 16


def _bf16r_bits(u):
    r = (u + jnp.int32(0x7FFF) + ((u >> 16) & 1)) & jnp.int32(-65536)
    return lax.bitcast_convert_type(r, jnp.float32)


_DN = lax.GatherDimensionNumbers(
    offset_dims=(), collapsed_slice_dims=(0,), start_index_map=(0,))


def _bcast(vec, l):
    idx = jnp.full((16, 1), l, jnp.int32)
    return lax.gather(vec, idx, _DN, slice_sizes=(1,),
                      mode=lax.GatherScatterMode.PROMISE_IN_BOUNDS)


def _edge_body(hl, pk3, wep, zrows, out,
               pk_v, rows_v, wep_v, agg_sh, gs0, gs1):
    c = lax.axis_index("c")
    s = lax.axis_index("s")
    wid = c * 16 + s
    pltpu.sync_copy(wep, wep_v)
    for j in range(16):
        wep_v[pl.ds(16 * j, 16)] = _bf16r(wep_v[pl.ds(16 * j, 16)])
    pltpu.sync_copy(zrows.at[pl.ds(0, ZR)], agg_sh.at[pl.ds(s * ZR, ZR)])
    plsc.subcore_barrier()

    gsems = (gs0, gs1)

    def gather(k, b):
        return pltpu.make_async_copy(
            hl.at[pk_v.at[b, 0]], rows_v.at[b], gsems[b])

    pltpu.sync_copy(pk3.at[wid, 0], pk_v.at[0])
    gather(0, 0).start()

    def step(k, b):
        nb = 1 - b

        @pl.when(k + 1 < NCH)
        def _():
            pltpu.sync_copy(pk3.at[wid, k + 1], pk_v.at[nb])
            gather(k + 1, nb).start()

        gather(k, b).wait()

        def group(g, carry2):
            av = _bf16r_bits(pk_v[b, 2, pl.ds(g * 16, 16)])
            bv = _bf16r_bits(pk_v[b, 3, pl.ds(g * 16, 16)])
            abc = [_bcast(av, l) for l in range(16)]
            bbc = [_bcast(bv, l) for l in range(16)]
            for j in range(8):
                w0 = wep_v[pl.ds(16 * j, 16)]
                w1 = wep_v[pl.ds(128 + 16 * j, 16)]
                bj = wep_v[pl.ds(256 + 16 * j, 16)]
                for l in range(16):
                    e = g * 16 + l
                    r = rows_v[b, e, pl.ds(16 * j, 16)]
                    rows_v[b, e, pl.ds(16 * j, 16)] = jnp.maximum(
                        r + (abc[l] * w0 + bbc[l] * w1 + bj), 0.0)
            return carry2

        lax.fori_loop(0, CH // 16, group, 0)
        pltpu.sync_copy(rows_v.at[b], agg_sh.at[pk_v.at[b, 1]], add=True)

    def pair(t, carry):
        step(t * 2, 0)
        step(t * 2 + 1, 1)
        return carry

    lax.fori_loop(0, NCH // 2, pair, 0)
    step(NCH - 1, 0)
    plsc.subcore_barrier()
    pltpu.sync_copy(agg_sh.at[pl.ds(s * ZR, ZR)],
                    out.at[c, pl.ds(s * ZR, ZR)])


def _edge_call(hl, pk3, wep, zrows):
    mesh = plsc.VectorSubcoreMesh(core_axis_name="c", subcore_axis_name="s")
    f = pl.kernel(
        _edge_body,
        mesh=mesh,
        out_type=jax.ShapeDtypeStruct((2, NP, EMB), jnp.float32),
        scratch_types=[
            pltpu.VMEM((2, 4, CH), jnp.int32),
            pltpu.VMEM((2, CH, EMB), jnp.float32),
            pltpu.VMEM((3 * EMB,), jnp.float32),
            pltpu.VMEM_SHARED((NP, EMB), jnp.float32),
            pltpu.SemaphoreType.DMA,
            pltpu.SemaphoreType.DMA,
        ],
    )
    return f(hl, pk3, wep, zrows)


def kernel(x, edge_index, edge_attr, node_depth, batch, params):
    p = params
    nd = jnp.clip(node_depth[:, 0], 0, 20).astype(jnp.int32)
    xc = jnp.stack([x[:, 0].astype(jnp.int32),
                    100 + x[:, 1].astype(jnp.int32),
                    200 + nd], axis=1)
    xc3 = xc.reshape(GRID, BLK, 3)
    batch3 = batch.astype(jnp.int32).reshape(GRID, 1, BLK)
    tbl = jnp.zeros((TBL, EMB), jnp.float32)
    tbl = tbl.at[0:100].set(p['type_emb'])
    tbl = tbl.at[100:200].set(p['attr_emb'])
    tbl = tbl.at[200:221].set(p['depth_emb'])
    vnrow = p['vn_emb'][0:1]
    src3 = edge_index[0].astype(jnp.int32).reshape(NW, NCH, CH)
    dst3 = edge_index[1].astype(jnp.int32).reshape(NW, NCH, CH)
    abits = lax.bitcast_convert_type(edge_attr[:, 0], jnp.int32)
    bbits = lax.bitcast_convert_type(edge_attr[:, 1], jnp.int32)
    pk3 = jnp.stack([src3, dst3, abits.reshape(NW, NCH, CH),
                     bbits.reshape(NW, NCH, CH)], axis=2)
    zrows = jnp.zeros((ZR, EMB), jnp.float32)

    vn = jnp.broadcast_to(vnrow, (NG, EMB))
    hl, pooled = _embed_call(xc3, batch3, tbl, vnrow)

    out = None
    for layer in range(3):
        cp = p['convs'][layer]
        web = cp['We']
        wep = jnp.concatenate([web[0], web[1], cp['be']])
        agg2 = _edge_call(hl, pk3, wep, zrows)
        epsp1 = (1.0 + cp['eps']).reshape(1, 1)
        y1, st1 = _mlp1_call(hl, agg2, epsp1, cp['W1'],
                             cp['b1'].reshape(1, 2 * EMB))
        y2, st2 = _mlp2_call(y1, st1, cp['g1'].reshape(1, 2 * EMB),
                             cp['bb1'].reshape(1, 2 * EMB), cp['W2'],
                             cp['b2'].reshape(1, EMB))
        g = p['bn_g'][layer].reshape(1, EMB)
        b = p['bn_b'][layer].reshape(1, EMB)
        if layer < 2:
            h = _bnout_call(y2, st2, g, b, relu=True)
            vn = _vnmlp_call(pooled, vn, p['vn_mlps'][layer])
            hl, pooled = _addvn_call(h, batch3, vn)
        else:
            out = _bnout_call(y2, st2, g, b, relu=False)
    return out


# speedup vs baseline: 4.3399x; 1.0015x over previous
import functools

import jax
import jax.numpy as jnp
from jax import lax
from jax.experimental import pallas as pl
from jax.experimental.pallas import tpu as pltpu
from jax.experimental.pallas import tpu_sc as plsc

N = 10000
E = 320000
EMB = 128
NG = 128
BLK = 1000
GRID = N // BLK
TBL = 256


def _embed_body(xc_ref, b_ref, tbl_ref, vnrow_ref, hl_ref, pooled_ref):
    i = pl.program_id(0)
    xc = xc_ref[0]
    cols = jax.lax.broadcasted_iota(jnp.int32, (BLK, TBL), 1)
    oh = ((cols == xc[:, 0:1]) | (cols == xc[:, 1:2]) | (cols == xc[:, 2:3]))
    oh = oh.astype(jnp.float32)
    hl = jnp.dot(oh, tbl_ref[...], preferred_element_type=jnp.float32, precision=jax.lax.Precision.HIGHEST)
    hl = hl + vnrow_ref[0][None, :]
    hl_ref[...] = hl
    seg = b_ref[0, 0]
    gcols = jax.lax.broadcasted_iota(jnp.int32, (BLK, NG), 1)
    ohb = (gcols == seg[:, None]).astype(jnp.float32)
    part = jax.lax.dot_general(ohb, hl, (((0,), (0,)), ((), ())),
                               preferred_element_type=jnp.float32, precision=jax.lax.Precision.HIGHEST)

    @pl.when(i == 0)
    def _():
        pooled_ref[...] = part

    @pl.when(i != 0)
    def _():
        pooled_ref[...] += part


def _embed_call(xc3, batch3, tbl, vnrow):
    return pl.pallas_call(
        _embed_body,
        grid=(GRID,),
        in_specs=[
            pl.BlockSpec((1, BLK, 3), lambda i: (i, 0, 0)),
            pl.BlockSpec((1, 1, BLK), lambda i: (i, 0, 0)),
            pl.BlockSpec((TBL, EMB), lambda i: (0, 0)),
            pl.BlockSpec((1, EMB), lambda i: (0, 0)),
        ],
        out_specs=[
            pl.BlockSpec((BLK, EMB), lambda i: (i, 0)),
            pl.BlockSpec((NG, EMB), lambda i: (0, 0)),
        ],
        out_shape=[
            jax.ShapeDtypeStruct((N, EMB), jnp.float32),
            jax.ShapeDtypeStruct((NG, EMB), jnp.float32),
        ],
    )(xc3, batch3, tbl, vnrow)


def _addvn_body(h_ref, b_ref, vn_ref, hl_ref, pooled_ref):
    i = pl.program_id(0)
    seg = b_ref[0, 0]
    gcols = jax.lax.broadcasted_iota(jnp.int32, (BLK, NG), 1)
    ohb = (gcols == seg[:, None]).astype(jnp.float32)
    hl = h_ref[...] + jnp.dot(ohb, vn_ref[...],
                              preferred_element_type=jnp.float32, precision=jax.lax.Precision.HIGHEST)
    hl_ref[...] = hl
    part = jax.lax.dot_general(ohb, hl, (((0,), (0,)), ((), ())),
                               preferred_element_type=jnp.float32, precision=jax.lax.Precision.HIGHEST)

    @pl.when(i == 0)
    def _():
        pooled_ref[...] = part

    @pl.when(i != 0)
    def _():
        pooled_ref[...] += part


def _addvn_call(h, batch3, vn):
    return pl.pallas_call(
        _addvn_body,
        grid=(GRID,),
        in_specs=[
            pl.BlockSpec((BLK, EMB), lambda i: (i, 0)),
            pl.BlockSpec((1, 1, BLK), lambda i: (i, 0, 0)),
            pl.BlockSpec((NG, EMB), lambda i: (0, 0)),
        ],
        out_specs=[
            pl.BlockSpec((BLK, EMB), lambda i: (i, 0)),
            pl.BlockSpec((NG, EMB), lambda i: (0, 0)),
        ],
        out_shape=[
            jax.ShapeDtypeStruct((N, EMB), jnp.float32),
            jax.ShapeDtypeStruct((NG, EMB), jnp.float32),
        ],
    )(h, batch3, vn)


def _mlp1_body(hl_ref, a0_ref, a1_ref, epsp1_ref, w1_ref, b1_ref,
               y1_ref, st_ref):
    i = pl.program_id(0)
    z = epsp1_ref[0, 0] * hl_ref[...] + a0_ref[0] + a1_ref[0]
    y1 = jnp.dot(z, w1_ref[...], preferred_element_type=jnp.float32, precision=jax.lax.Precision.DEFAULT)
    y1 = y1 + b1_ref[0][None, :]
    y1_ref[...] = y1
    s = jnp.sum(y1, axis=0)
    ss = jnp.sum(y1 * y1, axis=0)
    part = jnp.stack([s, ss])

    @pl.when(i == 0)
    def _():
        st_ref[...] = part

    @pl.when(i != 0)
    def _():
        st_ref[...] += part


def _mlp1_call(hl, agg2, epsp1, w1, b1):
    h2 = 2 * EMB
    return pl.pallas_call(
        _mlp1_body,
        grid=(GRID,),
        in_specs=[
            pl.BlockSpec((BLK, EMB), lambda i: (i, 0)),
            pl.BlockSpec((1, BLK, EMB), lambda i: (0, i, 0)),
            pl.BlockSpec((1, BLK, EMB), lambda i: (1, i, 0)),
            pl.BlockSpec((1, 1), lambda i: (0, 0)),
            pl.BlockSpec((EMB, h2), lambda i: (0, 0)),
            pl.BlockSpec((1, h2), lambda i: (0, 0)),
        ],
        out_specs=[
            pl.BlockSpec((BLK, h2), lambda i: (i, 0)),
            pl.BlockSpec((2, h2), lambda i: (0, 0)),
        ],
        out_shape=[
            jax.ShapeDtypeStruct((N, h2), jnp.float32),
            jax.ShapeDtypeStruct((2, h2), jnp.float32),
        ],
    )(hl, agg2, agg2, epsp1, w1, b1)


def _mlp2_body(y1_ref, st_ref, g1_ref, bb1_ref, w2_ref, b2_ref,
               y2_ref, st2_ref):
    i = pl.program_id(0)
    s = st_ref[0]
    ss = st_ref[1]
    m = s / N
    v = ss / N - m * m
    inv = 1.0 / jnp.sqrt(v + 1e-5)
    t = g1_ref[0][None, :] * (y1_ref[...] - m[None, :]) * inv[None, :] \
        + bb1_ref[0][None, :]
    t = jax.nn.relu(t)
    y2 = jnp.dot(t, w2_ref[...], preferred_element_type=jnp.float32, precision=jax.lax.Precision.DEFAULT)
    y2 = y2 + b2_ref[0][None, :]
    y2_ref[...] = y2
    part = jnp.stack([jnp.sum(y2, axis=0), jnp.sum(y2 * y2, axis=0)])

    @pl.when(i == 0)
    def _():
        st2_ref[...] = part

    @pl.when(i != 0)
    def _():
        st2_ref[...] += part


def _mlp2_call(y1, st, g1, bb1, w2, b2):
    h2 = 2 * EMB
    return pl.pallas_call(
        _mlp2_body,
        grid=(GRID,),
        in_specs=[
            pl.BlockSpec((BLK, h2), lambda i: (i, 0)),
            pl.BlockSpec((2, h2), lambda i: (0, 0)),
            pl.BlockSpec((1, h2), lambda i: (0, 0)),
            pl.BlockSpec((1, h2), lambda i: (0, 0)),
            pl.BlockSpec((h2, EMB), lambda i: (0, 0)),
            pl.BlockSpec((1, EMB), lambda i: (0, 0)),
        ],
        out_specs=[
            pl.BlockSpec((BLK, EMB), lambda i: (i, 0)),
            pl.BlockSpec((2, EMB), lambda i: (0, 0)),
        ],
        out_shape=[
            jax.ShapeDtypeStruct((N, EMB), jnp.float32),
            jax.ShapeDtypeStruct((2, EMB), jnp.float32),
        ],
    )(y1, st, g1, bb1, w2, b2)


def _bnout_body(relu, y2_ref, st_ref, g_ref, b_ref, h_ref):
    s = st_ref[0]
    ss = st_ref[1]
    m = s / N
    v = ss / N - m * m
    inv = 1.0 / jnp.sqrt(v + 1e-5)
    h = g_ref[0][None, :] * (y2_ref[...] - m[None, :]) * inv[None, :] \
        + b_ref[0][None, :]
    if relu:
        h = jax.nn.relu(h)
    h_ref[...] = h


def _bnout_call(y2, st, g, b, relu):
    return pl.pallas_call(
        functools.partial(_bnout_body, relu),
        grid=(GRID,),
        in_specs=[
            pl.BlockSpec((BLK, EMB), lambda i: (i, 0)),
            pl.BlockSpec((2, EMB), lambda i: (0, 0)),
            pl.BlockSpec((1, EMB), lambda i: (0, 0)),
            pl.BlockSpec((1, EMB), lambda i: (0, 0)),
        ],
        out_specs=pl.BlockSpec((BLK, EMB), lambda i: (i, 0)),
        out_shape=jax.ShapeDtypeStruct((N, EMB), jnp.float32),
    )(y2, st, g, b)


def _vnmlp_body(pooled_ref, vn_ref, w1_ref, b1_ref, g1_ref, bb1_ref,
                w2_ref, b2_ref, g2_ref, bb2_ref, out_ref):
    p = pooled_ref[...] + vn_ref[...]
    t = jnp.dot(p, w1_ref[...], preferred_element_type=jnp.float32, precision=jax.lax.Precision.DEFAULT)
    t = t + b1_ref[0][None, :]
    m = jnp.mean(t, axis=0)
    v = jnp.mean(t * t, axis=0) - m * m
    t = g1_ref[0] * (t - m[None, :]) / jnp.sqrt(v + 1e-5)[None, :] \
        + bb1_ref[0][None, :]
    t = jax.nn.relu(t)
    u = jnp.dot(t, w2_ref[...], preferred_element_type=jnp.float32, precision=jax.lax.Precision.DEFAULT)
    u = u + b2_ref[0][None, :]
    m2 = jnp.mean(u, axis=0)
    v2 = jnp.mean(u * u, axis=0) - m2 * m2
    u = g2_ref[0] * (u - m2[None, :]) / jnp.sqrt(v2 + 1e-5)[None, :] \
        + bb2_ref[0][None, :]
    out_ref[...] = jax.nn.relu(u)


def _vnmlp_call(pooled, vn, mp):
    h2 = 2 * EMB
    args = (pooled, vn, mp['W1'], mp['b1'].reshape(1, h2),
            mp['g1'].reshape(1, h2), mp['bb1'].reshape(1, h2),
            mp['W2'], mp['b2'].reshape(1, EMB),
            mp['g2'].reshape(1, EMB), mp['bb2'].reshape(1, EMB))
    return pl.pallas_call(
        _vnmlp_body,
        out_shape=jax.ShapeDtypeStruct((NG, EMB), jnp.float32),
    )(*args)


def _bf16r(v):
    u = lax.bitcast_convert_type(v, jnp.int32)
    r = (u + jnp.int32(0x7FFF) + ((u >> 16) & 1)) & jnp.int32(-65536)
    return lax.bitcast_convert_type(r, jnp.float32)


NW = 32
EPW = E // NW
CH = 80
NCH = EPW // CH
NP = 10240
ZR = NP // 16


def _bf16r_bits(u):
    r = (u + jnp.int32(0x7FFF) + ((u >> 16) & 1)) & jnp.int32(-65536)
    return lax.bitcast_convert_type(r, jnp.float32)


_DN = lax.GatherDimensionNumbers(
    offset_dims=(), collapsed_slice_dims=(0,), start_index_map=(0,))


def _bcast(vec, l):
    idx = jnp.full((16, 1), l, jnp.int32)
    return lax.gather(vec, idx, _DN, slice_sizes=(1,),
                      mode=lax.GatherScatterMode.PROMISE_IN_BOUNDS)


def _edge_body(hl, pk3, wep, zrows, out,
               pk_v, rows_v, wep_v, agg_sh, gs0, gs1, ss0, ss1):
    c = lax.axis_index("c")
    s = lax.axis_index("s")
    wid = c * 16 + s
    pltpu.sync_copy(wep, wep_v)
    for j in range(16):
        wep_v[pl.ds(16 * j, 16)] = _bf16r(wep_v[pl.ds(16 * j, 16)])
    pltpu.sync_copy(zrows.at[pl.ds(0, ZR)], agg_sh.at[pl.ds(s * ZR, ZR)])
    plsc.subcore_barrier()

    gsems = (gs0, gs1)
    ssems = (ss0, ss1)

    def gather(k, b):
        return pltpu.make_async_copy(
            hl.at[pk_v.at[b, 0]], rows_v.at[b], gsems[b])

    def scat(b):
        return pltpu.make_async_copy(
            rows_v.at[b], agg_sh.at[pk_v.at[b, 1]], ssems[b])

    pltpu.sync_copy(pk3.at[wid, 0], pk_v.at[0])
    gather(0, 0).start()

    def step(k, b):
        nb = 1 - b

        @pl.when(k + 1 < NCH)
        def _():
            @pl.when(k >= 1)
            def _():
                scat(nb).wait()

            pltpu.sync_copy(pk3.at[wid, k + 1], pk_v.at[nb])
            gather(k + 1, nb).start()

        gather(k, b).wait()

        def group(g, carry2):
            av = _bf16r_bits(pk_v[b, 2, pl.ds(g * 16, 16)])
            bv = _bf16r_bits(pk_v[b, 3, pl.ds(g * 16, 16)])
            abc = [_bcast(av, l) for l in range(16)]
            bbc = [_bcast(bv, l) for l in range(16)]
            for j in range(8):
                w0 = wep_v[pl.ds(16 * j, 16)]
                w1 = wep_v[pl.ds(128 + 16 * j, 16)]
                bj = wep_v[pl.ds(256 + 16 * j, 16)]
                for l in range(16):
                    e = g * 16 + l
                    r = rows_v[b, e, pl.ds(16 * j, 16)]
                    rows_v[b, e, pl.ds(16 * j, 16)] = jnp.maximum(
                        r + (abc[l] * w0 + bbc[l] * w1 + bj), 0.0)
            return carry2

        lax.fori_loop(0, CH // 16, group, 0)
        pltpu.async_copy(rows_v.at[b], agg_sh.at[pk_v.at[b, 1]], ssems[b],
                         add=True)

    def pair(t, carry):
        step(t * 2, 0)
        step(t * 2 + 1, 1)
        return carry

    lax.fori_loop(0, NCH // 2, pair, 0)
    step(NCH - 1, 0)
    scat(1).wait()
    scat(0).wait()
    plsc.subcore_barrier()
    pltpu.sync_copy(agg_sh.at[pl.ds(s * ZR, ZR)],
                    out.at[c, pl.ds(s * ZR, ZR)])


def _edge_call(hl, pk3, wep, zrows):
    mesh = plsc.VectorSubcoreMesh(core_axis_name="c", subcore_axis_name="s")
    f = pl.kernel(
        _edge_body,
        mesh=mesh,
        out_type=jax.ShapeDtypeStruct((2, NP, EMB), jnp.float32),
        scratch_types=[
            pltpu.VMEM((2, 4, CH), jnp.int32),
            pltpu.VMEM((2, CH, EMB), jnp.float32),
            pltpu.VMEM((3 * EMB,), jnp.float32),
            pltpu.VMEM_SHARED((NP, EMB), jnp.float32),
            pltpu.SemaphoreType.DMA,
            pltpu.SemaphoreType.DMA,
            pltpu.SemaphoreType.DMA,
            pltpu.SemaphoreType.DMA,
        ],
    )
    return f(hl, pk3, wep, zrows)


def kernel(x, edge_index, edge_attr, node_depth, batch, params):
    p = params
    nd = jnp.clip(node_depth[:, 0], 0, 20).astype(jnp.int32)
    xc = jnp.stack([x[:, 0].astype(jnp.int32),
                    100 + x[:, 1].astype(jnp.int32),
                    200 + nd], axis=1)
    xc3 = xc.reshape(GRID, BLK, 3)
    batch3 = batch.astype(jnp.int32).reshape(GRID, 1, BLK)
    tbl = jnp.zeros((TBL, EMB), jnp.float32)
    tbl = tbl.at[0:100].set(p['type_emb'])
    tbl = tbl.at[100:200].set(p['attr_emb'])
    tbl = tbl.at[200:221].set(p['depth_emb'])
    vnrow = p['vn_emb'][0:1]
    src3 = edge_index[0].astype(jnp.int32).reshape(NW, NCH, CH)
    dst3 = edge_index[1].astype(jnp.int32).reshape(NW, NCH, CH)
    abits = lax.bitcast_convert_type(edge_attr[:, 0], jnp.int32)
    bbits = lax.bitcast_convert_type(edge_attr[:, 1], jnp.int32)
    pk3 = jnp.stack([src3, dst3, abits.reshape(NW, NCH, CH),
                     bbits.reshape(NW, NCH, CH)], axis=2)
    zrows = jnp.zeros((ZR, EMB), jnp.float32)

    vn = jnp.broadcast_to(vnrow, (NG, EMB))
    hl, pooled = _embed_call(xc3, batch3, tbl, vnrow)

    out = None
    for layer in range(3):
        cp = p['convs'][layer]
        web = cp['We']
        wep = jnp.concatenate([web[0], web[1], cp['be']])
        agg2 = _edge_call(hl, pk3, wep, zrows)
        epsp1 = (1.0 + cp['eps']).reshape(1, 1)
        y1, st1 = _mlp1_call(hl, agg2, epsp1, cp['W1'],
                             cp['b1'].reshape(1, 2 * EMB))
        y2, st2 = _mlp2_call(y1, st1, cp['g1'].reshape(1, 2 * EMB),
                             cp['bb1'].reshape(1, 2 * EMB), cp['W2'],
                             cp['b2'].reshape(1, EMB))
        g = p['bn_g'][layer].reshape(1, EMB)
        b = p['bn_b'][layer].reshape(1, EMB)
        if layer < 2:
            h = _bnout_call(y2, st2, g, b, relu=True)
            vn = _vnmlp_call(pooled, vn, p['vn_mlps'][layer])
            hl, pooled = _addvn_call(h, batch3, vn)
        else:
            out = _bnout_call(y2, st2, g, b, relu=False)
    return out
